# baseline (speedup 1.0000x reference)
import functools

import jax
import jax.numpy as jnp
from jax import lax
from jax.experimental import pallas as pl
from jax.experimental.pallas import tpu as pltpu
from jax.experimental.pallas import tpu_sc as plsc

N = 1 << 21
RES = 128
C = 1024
R = N // C
BR = 32
NBLK = R // BR

_KS0 = 0
_KS1 = 42
_KS2 = _KS0 ^ _KS1 ^ 0x1BD11BDA
_ROTS = ((13, 15, 26, 6), (17, 29, 16, 24))


def _rotl(v, r):
    return (v << jnp.uint32(r)) | (v >> jnp.uint32(32 - r))


def _threefry_bits(g):
    ks = (jnp.uint32(_KS0), jnp.uint32(_KS1), jnp.uint32(_KS2))
    x0 = jnp.zeros_like(g) + ks[0]
    x1 = g + ks[1]
    for r in range(5):
        for d in _ROTS[r % 2]:
            x0 = x0 + x1
            x1 = _rotl(x1, d)
            x1 = x0 ^ x1
        x0 = x0 + ks[(r + 1) % 3]
        x1 = x1 + ks[(r + 2) % 3] + jnp.uint32(r + 1)
    return x0 ^ x1


def _tc1_body(xt_ref, lin_ref, thr_ref):
    i = pl.program_id(0)
    x = xt_ref[0]
    y = xt_ref[1]
    z = xt_ref[2]

    def vox(v):
        f = jnp.round((v + 1.0) * 128.0 * 0.5 - 0.5)
        return jnp.clip(f, 0.0, 127.0).astype(jnp.int32)

    lin_ref[...] = vox(z) * (RES * RES) + vox(y) * RES + vox(x)

    inb = ((x >= -1.0) & (x <= 1.0) & (y >= -1.0) & (y <= 1.0)
           & (z >= -1.0) & (z <= 1.0))

    row = lax.broadcasted_iota(jnp.uint32, (BR, C), 0)
    col = lax.broadcasted_iota(jnp.uint32, (BR, C), 1)
    g = (jnp.uint32(BR).astype(jnp.uint32) * i.astype(jnp.uint32) + row) \
        * jnp.uint32(C) + col
    bits = _threefry_bits(g)
    fb = (bits >> jnp.uint32(9)) | jnp.uint32(0x3F800000)
    u = lax.bitcast_convert_type(fb, jnp.float32) - 1.0
    t_u = -jnp.log1p(-u) - 1e-4
    thr = jnp.minimum(jnp.float32(0.01), t_u)
    thr_ref[...] = jnp.where(inb, thr, jnp.float32(jnp.inf))


_tc1 = pl.pallas_call(
    _tc1_body,
    grid=(NBLK,),
    in_specs=[pl.BlockSpec((3, BR, C), lambda i: (0, i, 0))],
    out_specs=[pl.BlockSpec((BR, C), lambda i: (i, 0)),
               pl.BlockSpec((BR, C), lambda i: (i, 0))],
    out_shape=[jax.ShapeDtypeStruct((R, C), jnp.int32),
               jax.ShapeDtypeStruct((R, C), jnp.float32)],
)


def _tc2_body(d_ref, thr_ref, o_ref):
    o_ref[...] = d_ref[...] > thr_ref[...]


_tc2 = pl.pallas_call(
    _tc2_body,
    grid=(NBLK,),
    in_specs=[pl.BlockSpec((BR, C), lambda i: (i, 0)),
              pl.BlockSpec((BR, C), lambda i: (i, 0))],
    out_specs=pl.BlockSpec((BR, C), lambda i: (i, 0)),
    out_shape=jax.ShapeDtypeStruct((R, C), jnp.bool_),
)

_NC = 2
_NS = 16
_NW = _NC * _NS
_NP = N // _NW
_CH = 8192
_NCH = _NP // _CH

@functools.cache
def _make_sc_gather():
    mesh = plsc.VectorSubcoreMesh(core_axis_name="c", subcore_axis_name="s")

    @functools.partial(
        pl.kernel,
        mesh=mesh,
        out_type=jax.ShapeDtypeStruct((N,), jnp.float32),
        scratch_types=[pltpu.VMEM((_CH,), jnp.int32),
                       pltpu.VMEM((_CH,), jnp.float32),
                       pltpu.SemaphoreType.DMA],
    )
    def _sc_gather(lin_hbm, grid_hbm, d_hbm, idx_v, d_v, sem):
        wid = lax.axis_index("s") * _NC + lax.axis_index("c")
        base0 = wid * _NP
        for k in range(_NCH):
            base = base0 + k * _CH
            pltpu.sync_copy(lin_hbm.at[pl.ds(base, _CH)], idx_v)
            pltpu.async_copy(grid_hbm.at[idx_v], d_v, sem).wait()
            pltpu.sync_copy(d_v, d_hbm.at[pl.ds(base, _CH)])

    return _sc_gather


def kernel(xyz_ndc, grid):
    xt = jnp.transpose(xyz_ndc).reshape(3, R, C)
    lin, thr = _tc1(xt)
    d = _make_sc_gather()(lin.reshape(N), grid.reshape(N))
    out = _tc2(d.reshape(R, C), thr)
    return out.reshape(N)


# speedup vs baseline: 1.3773x; 1.3773x over previous
import functools

import jax
import jax.numpy as jnp
from jax import lax
from jax.experimental import pallas as pl
from jax.experimental.pallas import tpu as pltpu
from jax.experimental.pallas import tpu_sc as plsc

N = 1 << 21
RES = 128
C = 128
R = N // C
BR = 512
NBLK = R // BR

_KS0 = 0
_KS1 = 42
_KS2 = _KS0 ^ _KS1 ^ 0x1BD11BDA
_ROTS = ((13, 15, 26, 6), (17, 29, 16, 24))


def _rotl(v, r):
    return (v << jnp.uint32(r)) | (v >> jnp.uint32(32 - r))


def _threefry_bits(g):
    ks = (jnp.uint32(_KS0), jnp.uint32(_KS1), jnp.uint32(_KS2))
    x0 = jnp.zeros_like(g) + ks[0]
    x1 = g + ks[1]
    for r in range(5):
        for d in _ROTS[r % 2]:
            x0 = x0 + x1
            x1 = _rotl(x1, d)
            x1 = x0 ^ x1
        x0 = x0 + ks[(r + 1) % 3]
        x1 = x1 + ks[(r + 2) % 3] + jnp.uint32(r + 1)
    return x0 ^ x1


def _tca_body(xt_ref, lin_ref, inb_ref):
    x = xt_ref[0]
    y = xt_ref[1]
    z = xt_ref[2]

    def vox(v):
        f = jnp.round((v + 1.0) * 128.0 * 0.5 - 0.5)
        return jnp.clip(f, 0.0, 127.0).astype(jnp.int32)

    lin_ref[...] = vox(z) * (RES * RES) + vox(y) * RES + vox(x)
    inb = ((x >= -1.0) & (x <= 1.0) & (y >= -1.0) & (y <= 1.0)
           & (z >= -1.0) & (z <= 1.0))
    inb_ref[...] = inb.astype(jnp.uint8)


_tca = pl.pallas_call(
    _tca_body,
    grid=(NBLK,),
    in_specs=[pl.BlockSpec((3, BR, C), lambda i: (0, i, 0))],
    out_specs=[pl.BlockSpec((BR, C), lambda i: (i, 0)),
               pl.BlockSpec((BR, C), lambda i: (i, 0))],
    out_shape=[jax.ShapeDtypeStruct((R, C), jnp.int32),
               jax.ShapeDtypeStruct((R, C), jnp.uint8)],
)


def _tcb_body(inb_ref, thr_ref):
    i = pl.program_id(0)
    row = lax.broadcasted_iota(jnp.uint32, (BR, C), 0)
    col = lax.broadcasted_iota(jnp.uint32, (BR, C), 1)
    g = (jnp.uint32(BR) * i.astype(jnp.uint32) + row) * jnp.uint32(C) + col
    bits = _threefry_bits(g)
    fb = (bits >> jnp.uint32(9)) | jnp.uint32(0x3F800000)
    u = lax.bitcast_convert_type(fb, jnp.float32) - 1.0
    t_u = -jnp.log1p(-u) - 1e-4
    thr = jnp.minimum(jnp.float32(0.01), t_u)
    inb = inb_ref[...] != 0
    thr_ref[...] = jnp.where(inb, thr, jnp.float32(jnp.inf))


_tcb = pl.pallas_call(
    _tcb_body,
    grid=(NBLK,),
    in_specs=[pl.BlockSpec((BR, C), lambda i: (i, 0))],
    out_specs=pl.BlockSpec((BR, C), lambda i: (i, 0)),
    out_shape=jax.ShapeDtypeStruct((R, C), jnp.float32),
)


def _tcc_body(d_ref, thr_ref, o_ref):
    o_ref[...] = (d_ref[...] > thr_ref[...]).astype(jnp.uint8)


_tcc = pl.pallas_call(
    _tcc_body,
    grid=(NBLK,),
    in_specs=[pl.BlockSpec((BR, C), lambda i: (i, 0)),
              pl.BlockSpec((BR, C), lambda i: (i, 0))],
    out_specs=pl.BlockSpec((BR, C), lambda i: (i, 0)),
    out_shape=jax.ShapeDtypeStruct((R, C), jnp.uint8),
)

_NC = 2
_NS = 16
_NW = _NC * _NS
_NP = N // _NW
_CH = 8192
_NCH = _NP // _CH


@functools.cache
def _make_sc_gather():
    mesh = plsc.VectorSubcoreMesh(core_axis_name="c", subcore_axis_name="s")

    @functools.partial(
        pl.kernel,
        mesh=mesh,
        out_type=jax.ShapeDtypeStruct((N,), jnp.float32),
        scratch_types=[pltpu.VMEM((_CH,), jnp.int32),
                       pltpu.VMEM((_CH,), jnp.float32),
                       pltpu.SemaphoreType.DMA],
    )
    def _sc_gather(lin_hbm, grid_hbm, d_hbm, idx_v, d_v, sem):
        wid = lax.axis_index("s") * _NC + lax.axis_index("c")
        base0 = wid * _NP
        for k in range(_NCH):
            base = base0 + k * _CH
            pltpu.sync_copy(lin_hbm.at[pl.ds(base, _CH)], idx_v)
            pltpu.async_copy(grid_hbm.at[idx_v], d_v, sem).wait()
            pltpu.sync_copy(d_v, d_hbm.at[pl.ds(base, _CH)])

    return _sc_gather


def kernel(xyz_ndc, grid):
    xt = jnp.transpose(xyz_ndc).reshape(3, R, C)
    lin, inb = _tca(xt)
    d = _make_sc_gather()(lin.reshape(N), grid.reshape(N))
    thr = _tcb(inb)
    out8 = _tcc(d.reshape(R, C), thr)
    return out8.reshape(N).astype(jnp.bool_)


# speedup vs baseline: 1.4387x; 1.0446x over previous
import functools

import jax
import jax.numpy as jnp
from jax import lax
from jax.experimental import pallas as pl
from jax.experimental.pallas import tpu as pltpu
from jax.experimental.pallas import tpu_sc as plsc

N = 1 << 21
RES = 128
C = 128
R = N // C
BR = 512
NBLK = R // BR

_KS0 = 0
_KS1 = 42
_KS2 = _KS0 ^ _KS1 ^ 0x1BD11BDA
_ROTS = ((13, 15, 26, 6), (17, 29, 16, 24))


def _rotl(v, r):
    return (v << jnp.uint32(r)) | (v >> jnp.uint32(32 - r))


def _threefry_bits(g):
    ks = (jnp.uint32(_KS0), jnp.uint32(_KS1), jnp.uint32(_KS2))
    x0 = jnp.zeros_like(g) + ks[0]
    x1 = g + ks[1]
    for r in range(5):
        for d in _ROTS[r % 2]:
            x0 = x0 + x1
            x1 = _rotl(x1, d)
            x1 = x0 ^ x1
        x0 = x0 + ks[(r + 1) % 3]
        x1 = x1 + ks[(r + 2) % 3] + jnp.uint32(r + 1)
    return x0 ^ x1


def _tca_body(xt_ref, lin_ref, inb_ref):
    x = xt_ref[0]
    y = xt_ref[1]
    z = xt_ref[2]

    def vox(v):
        f = jnp.round((v + 1.0) * 128.0 * 0.5 - 0.5)
        return jnp.clip(f, 0.0, 127.0).astype(jnp.int32)

    lin_ref[...] = vox(z) * (RES * RES) + vox(y) * RES + vox(x)
    inb = ((x >= -1.0) & (x <= 1.0) & (y >= -1.0) & (y <= 1.0)
           & (z >= -1.0) & (z <= 1.0))
    inb_ref[...] = inb.astype(jnp.uint8)


_tca = pl.pallas_call(
    _tca_body,
    grid=(NBLK,),
    in_specs=[pl.BlockSpec((3, BR, C), lambda i: (0, i, 0))],
    out_specs=[pl.BlockSpec((BR, C), lambda i: (i, 0)),
               pl.BlockSpec((BR, C), lambda i: (i, 0))],
    out_shape=[jax.ShapeDtypeStruct((R, C), jnp.int32),
               jax.ShapeDtypeStruct((R, C), jnp.uint8)],
)


def _tcb_body(inb_ref, thr_ref):
    i = pl.program_id(0)
    row = lax.broadcasted_iota(jnp.uint32, (BR, C), 0)
    col = lax.broadcasted_iota(jnp.uint32, (BR, C), 1)
    g = (jnp.uint32(BR) * i.astype(jnp.uint32) + row) * jnp.uint32(C) + col
    bits = _threefry_bits(g)
    fb = (bits >> jnp.uint32(9)) | jnp.uint32(0x3F800000)
    u = lax.bitcast_convert_type(fb, jnp.float32) - 1.0
    t_u = -jnp.log1p(-u) - 1e-4
    thr = jnp.minimum(jnp.float32(0.01), t_u)
    inb = inb_ref[...] != 0
    thr_ref[...] = jnp.where(inb, thr, jnp.float32(jnp.inf))


_tcb = pl.pallas_call(
    _tcb_body,
    grid=(NBLK,),
    in_specs=[pl.BlockSpec((BR, C), lambda i: (i, 0))],
    out_specs=pl.BlockSpec((BR, C), lambda i: (i, 0)),
    out_shape=jax.ShapeDtypeStruct((R, C), jnp.float32),
)


def _tcc_body(d_ref, thr_ref, o_ref):
    o_ref[...] = (d_ref[...] > thr_ref[...]).astype(jnp.uint8)


_tcc = pl.pallas_call(
    _tcc_body,
    grid=(NBLK,),
    in_specs=[pl.BlockSpec((BR, C), lambda i: (i, 0)),
              pl.BlockSpec((BR, C), lambda i: (i, 0))],
    out_specs=pl.BlockSpec((BR, C), lambda i: (i, 0)),
    out_shape=jax.ShapeDtypeStruct((R, C), jnp.uint8),
)

_NC = 2
_NS = 16
_NW = _NC * _NS
_NP = N // _NW
_CH = 16384
_NCH = _NP // _CH


@functools.cache
def _make_sc_gather():
    mesh = plsc.VectorSubcoreMesh(core_axis_name="c", subcore_axis_name="s")

    @functools.partial(
        pl.kernel,
        mesh=mesh,
        out_type=jax.ShapeDtypeStruct((N,), jnp.float32),
        scratch_types=[pltpu.VMEM((_CH,), jnp.int32),
                       pltpu.VMEM((_CH,), jnp.int32),
                       pltpu.VMEM((_CH,), jnp.float32),
                       pltpu.VMEM((_CH,), jnp.float32),
                       pltpu.SemaphoreType.DMA,
                       pltpu.SemaphoreType.DMA],
    )
    def _sc_gather(lin_hbm, grid_hbm, d_hbm, idx0, idx1, d0, d1, s0, s1):
        wid = lax.axis_index("s") * _NC + lax.axis_index("c")
        base0 = wid * _NP
        idxs, ds, sems = (idx0, idx1), (d0, d1), (s0, s1)
        pltpu.sync_copy(lin_hbm.at[pl.ds(base0, _CH)], idx0)
        cps = [pltpu.async_copy(grid_hbm.at[idx0], d0, s0)]
        for k in range(1, _NCH):
            b = k % 2
            pltpu.sync_copy(lin_hbm.at[pl.ds(base0 + k * _CH, _CH)], idxs[b])
            cps.append(pltpu.async_copy(grid_hbm.at[idxs[b]], ds[b], sems[b]))
            cps[k - 1].wait()
            pltpu.sync_copy(ds[1 - b], d_hbm.at[pl.ds(base0 + (k - 1) * _CH, _CH)])
        cps[_NCH - 1].wait()
        pltpu.sync_copy(ds[(_NCH - 1) % 2],
                        d_hbm.at[pl.ds(base0 + (_NCH - 1) * _CH, _CH)])

    return _sc_gather


def kernel(xyz_ndc, grid):
    xt = jnp.transpose(xyz_ndc).reshape(3, R, C)
    lin, inb = _tca(xt)
    d = _make_sc_gather()(lin.reshape(N), grid.reshape(N))
    thr = _tcb(inb)
    out8 = _tcc(d.reshape(R, C), thr)
    return out8.reshape(N).astype(jnp.bool_)


# speedup vs baseline: 1.4893x; 1.0351x over previous
import functools

import jax
import jax.numpy as jnp
from jax import lax
from jax.experimental import pallas as pl
from jax.experimental.pallas import tpu as pltpu
from jax.experimental.pallas import tpu_sc as plsc

N = 1 << 21
RES = 128
C = 128
R = N // C
BR = 512
NBLK = R // BR

_KS0 = 0
_KS1 = 42
_KS2 = _KS0 ^ _KS1 ^ 0x1BD11BDA
_ROTS = ((13, 15, 26, 6), (17, 29, 16, 24))


def _rotl(v, r):
    return (v << jnp.uint32(r)) | (v >> jnp.uint32(32 - r))


def _threefry_bits(g):
    ks = (jnp.uint32(_KS0), jnp.uint32(_KS1), jnp.uint32(_KS2))
    x0 = jnp.zeros_like(g) + ks[0]
    x1 = g + ks[1]
    for r in range(5):
        for d in _ROTS[r % 2]:
            x0 = x0 + x1
            x1 = _rotl(x1, d)
            x1 = x0 ^ x1
        x0 = x0 + ks[(r + 1) % 3]
        x1 = x1 + ks[(r + 2) % 3] + jnp.uint32(r + 1)
    return x0 ^ x1


def _tcb_body(thr_ref):
    i = pl.program_id(0)
    row = lax.broadcasted_iota(jnp.uint32, (BR, C), 0)
    col = lax.broadcasted_iota(jnp.uint32, (BR, C), 1)
    g = (jnp.uint32(BR) * i.astype(jnp.uint32) + row) * jnp.uint32(C) + col
    bits = _threefry_bits(g)
    fb = (bits >> jnp.uint32(9)) | jnp.uint32(0x3F800000)
    u = lax.bitcast_convert_type(fb, jnp.float32) - 1.0
    t_u = -jnp.log1p(-u) - 1e-4
    thr_ref[...] = jnp.minimum(jnp.float32(0.01), t_u)


_tcb = pl.pallas_call(
    _tcb_body,
    grid=(NBLK,),
    out_specs=pl.BlockSpec((BR, C), lambda i: (i, 0)),
    out_shape=jax.ShapeDtypeStruct((R, C), jnp.float32),
)


def _tcc_body(d_ref, thr_ref, o_ref):
    o_ref[...] = (d_ref[...] > thr_ref[...]).astype(jnp.uint8)


_tcc = pl.pallas_call(
    _tcc_body,
    grid=(NBLK,),
    in_specs=[pl.BlockSpec((BR, C), lambda i: (i, 0)),
              pl.BlockSpec((BR, C), lambda i: (i, 0))],
    out_specs=pl.BlockSpec((BR, C), lambda i: (i, 0)),
    out_shape=jax.ShapeDtypeStruct((R, C), jnp.uint8),
)

_NC = 2
_NS = 16
_NW = _NC * _NS
_NP = N // _NW
_CH = 8192
_CROWS = _CH // C
_NCH = _NP // _CH
_RM = 12582912.0


def _voxf(v):
    f = (v + 1.0) * 64.0 - 0.5
    f = (f + _RM) - _RM
    return jnp.minimum(jnp.maximum(f, 0.0), 127.0)


def _sc_compute_chunk(xv, yv, zv, idx_v, pen_v):
    def body(i, _):
        row = i >> 3
        co = (i & 7) << 4
        p = i << 4
        x = xv[row, pl.ds(co, 16)]
        y = yv[row, pl.ds(co, 16)]
        z = zv[row, pl.ds(co, 16)]
        linf = (_voxf(z) * 128.0 + _voxf(y)) * 128.0 + _voxf(x)
        idx_v[pl.ds(p, 16)] = linf.astype(jnp.int32)
        inb = ((x >= -1.0) & (x <= 1.0) & (y >= -1.0) & (y <= 1.0)
               & (z >= -1.0) & (z <= 1.0))
        pen_v[pl.ds(p, 16)] = jnp.where(inb, jnp.zeros_like(x),
                                        jnp.full_like(x, -2.0))
        return 0

    lax.fori_loop(0, _CH // 16, body, 0)


def _sc_pass2(d_v, pen_v):
    def body(i, _):
        p = i << 4
        d_v[pl.ds(p, 16)] = d_v[pl.ds(p, 16)] + pen_v[pl.ds(p, 16)]
        return 0

    lax.fori_loop(0, _CH // 16, body, 0)


@functools.cache
def _make_sc_gather():
    mesh = plsc.VectorSubcoreMesh(core_axis_name="c", subcore_axis_name="s")

    slab = pltpu.VMEM((_CROWS, C), jnp.float32)
    flat_i = pltpu.VMEM((_CH,), jnp.int32)
    flat_f = pltpu.VMEM((_CH,), jnp.float32)

    @functools.partial(
        pl.kernel,
        mesh=mesh,
        out_type=jax.ShapeDtypeStruct((N,), jnp.float32),
        scratch_types=[slab, slab, slab, slab, slab, slab,
                       flat_i, flat_i, flat_f, flat_f, flat_f, flat_f,
                       pltpu.SemaphoreType.DMA, pltpu.SemaphoreType.DMA],
    )
    def _sc_gather(xt_hbm, grid_hbm, d_hbm,
                   xv0, yv0, zv0, xv1, yv1, zv1,
                   idx0, idx1, pen0, pen1, dv0, dv1, s0, s1):
        wid = lax.axis_index("s") * _NC + lax.axis_index("c")
        row0 = wid * (_NP // C)
        xvs, yvs, zvs = (xv0, xv1), (yv0, yv1), (zv0, zv1)
        idxs, pens, dvs, sems = (idx0, idx1), (pen0, pen1), (dv0, dv1), (s0, s1)

        def load_and_compute(k, b):
            r = row0 + k * _CROWS
            pltpu.sync_copy(xt_hbm.at[0, pl.ds(r, _CROWS), :], xvs[b])
            pltpu.sync_copy(xt_hbm.at[1, pl.ds(r, _CROWS), :], yvs[b])
            pltpu.sync_copy(xt_hbm.at[2, pl.ds(r, _CROWS), :], zvs[b])
            _sc_compute_chunk(xvs[b], yvs[b], zvs[b], idxs[b], pens[b])

        def drain(k):
            b = k % 2
            _sc_pass2(dvs[b], pens[b])
            pltpu.sync_copy(dvs[b],
                            d_hbm.at[pl.ds(wid * _NP + k * _CH, _CH)])

        load_and_compute(0, 0)
        cps = [pltpu.async_copy(grid_hbm.at[idx0], dv0, s0)]
        for k in range(1, _NCH):
            b = k % 2
            load_and_compute(k, b)
            cps.append(pltpu.async_copy(grid_hbm.at[idxs[b]], dvs[b], sems[b]))
            cps[k - 1].wait()
            drain(k - 1)
        cps[_NCH - 1].wait()
        drain(_NCH - 1)

    return _sc_gather


def kernel(xyz_ndc, grid):
    xt = jnp.transpose(xyz_ndc).reshape(3, R, C)
    d = _make_sc_gather()(xt, grid.reshape(N))
    thr = _tcb()
    out8 = _tcc(d.reshape(R, C), thr)
    return out8.reshape(N).view(jnp.bool_)
